# baseline (speedup 1.0000x reference)
import functools
import math

import jax
import jax.numpy as jnp
from jax import lax
from jax.experimental import pallas as pl
from jax.experimental.pallas import tpu as pltpu
from jax.experimental.pallas import tpu_sc as plsc

_LANES = 16
_NUM_WORKERS = 32


def kernel(input_ids, token_table, pos_table):
    B, S = input_ids.shape
    V, D = token_table.shape
    N = B * S
    scale = math.sqrt(float(D))
    s_per_w = S // _NUM_WORKERS
    nvec = D // _LANES

    mesh = plsc.VectorSubcoreMesh(core_axis_name="c", subcore_axis_name="s")

    @functools.partial(
        pl.kernel,
        mesh=mesh,
        out_type=jax.ShapeDtypeStruct((N, D), jnp.float32),
        scratch_types=[
            pltpu.VMEM((B, s_per_w), jnp.int32),
            pltpu.VMEM((s_per_w, D), jnp.float32),
            pltpu.VMEM((2 * s_per_w, D), jnp.float32),
            pltpu.SemaphoreType.DMA((2,)),
            pltpu.SemaphoreType.DMA((2,)),
            pltpu.SemaphoreType.DMA,
            pltpu.SemaphoreType.DMA,
        ],
    )
    def body(ids_hbm, tok_hbm, pos_hbm, out_hbm, idx_v, pos_v, tbuf,
             gsem, osem, isem, psem):
        wid = lax.axis_index("s") * 2 + lax.axis_index("c")
        s0 = wid * s_per_w
        idx_cps = [
            pltpu.async_copy(ids_hbm.at[pl.ds(b * S + s0, s_per_w)],
                             idx_v.at[b], isem)
            for b in range(B)
        ]
        pos_cp = pltpu.async_copy(pos_hbm.at[pl.ds(s0, s_per_w)], pos_v, psem)
        for cp in idx_cps:
            cp.wait()

        def slot_ref(slot):
            return tbuf.at[pl.ds(slot * s_per_w, s_per_w)]

        pltpu.async_copy(tok_hbm.at[idx_v.at[0]], slot_ref(0), gsem.at[0])
        pos_cp.wait()

        def chunk(b, _):
            slot = lax.rem(b, 2)
            nslot = 1 - slot

            @pl.when(b + 1 < B)
            def _():
                @pl.when(b >= 1)
                def _():
                    pltpu.make_async_copy(
                        slot_ref(nslot), out_hbm.at[pl.ds(0, s_per_w)],
                        osem.at[nslot]).wait()
                pltpu.async_copy(
                    tok_hbm.at[idx_v.at[b + 1]], slot_ref(nslot),
                    gsem.at[nslot])

            pltpu.make_async_copy(
                tok_hbm.at[pl.ds(0, s_per_w)], slot_ref(slot),
                gsem.at[slot]).wait()
            base = slot * s_per_w

            def row(r, _):
                for k in range(nvec):
                    sl = pl.ds(k * _LANES, _LANES)
                    tbuf[base + r, sl] = (
                        tbuf[base + r, sl] * scale + pos_v[r, sl])
                return 0

            lax.fori_loop(0, s_per_w, row, 0)
            pltpu.async_copy(
                slot_ref(slot), out_hbm.at[pl.ds(b * S + s0, s_per_w)],
                osem.at[slot])
            return 0

        lax.fori_loop(0, B, chunk, 0)
        for slot in range(2):
            pltpu.make_async_copy(
                slot_ref(slot), out_hbm.at[pl.ds(0, s_per_w)],
                osem.at[slot]).wait()

    out = body(input_ids.reshape(N), token_table, pos_table)
    return out.reshape(B, S, D)


# speedup vs baseline: 1.7199x; 1.7199x over previous
import functools
import math

import jax
import jax.numpy as jnp
from jax import lax
from jax.experimental import pallas as pl
from jax.experimental.pallas import tpu as pltpu
from jax.experimental.pallas import tpu_sc as plsc

_LANES = 16
_NUM_WORKERS = 32


def kernel(input_ids, token_table, pos_table):
    B, S = input_ids.shape
    V, D = token_table.shape
    N = B * S
    scale = math.sqrt(float(D))
    s_per_w = S // _NUM_WORKERS
    nvec = D // _LANES

    mesh = plsc.VectorSubcoreMesh(core_axis_name="c", subcore_axis_name="s")

    @functools.partial(
        pl.kernel,
        mesh=mesh,
        out_type=jax.ShapeDtypeStruct((N, D), jnp.float32),
        scratch_types=[
            pltpu.VMEM((B, s_per_w), jnp.int32),
            pltpu.VMEM((s_per_w, D), jnp.float32),
            pltpu.VMEM((s_per_w, D), jnp.float32),
            pltpu.VMEM((s_per_w, D), jnp.float32),
            pltpu.SemaphoreType.DMA,
            pltpu.SemaphoreType.DMA,
            pltpu.SemaphoreType.DMA,
            pltpu.SemaphoreType.DMA,
            pltpu.SemaphoreType.DMA,
            pltpu.SemaphoreType.DMA,
        ],
    )
    def body(ids_hbm, tok_hbm, pos_hbm, out_hbm, idx_v, pos_v, t0, t1,
             g0, g1, o0, o1, isem, psem):
        wid = lax.axis_index("s") * 2 + lax.axis_index("c")
        s0 = wid * s_per_w
        idx_cps = [
            pltpu.async_copy(ids_hbm.at[pl.ds(b * S + s0, s_per_w)],
                             idx_v.at[b], isem)
            for b in range(B)
        ]
        pos_cp = pltpu.async_copy(pos_hbm.at[pl.ds(s0, s_per_w)], pos_v, psem)
        for cp in idx_cps:
            cp.wait()

        tbufs = [t0, t1]
        gsems = [g0, g1]
        osems = [o0, o1]
        gathers = [None, None]
        stores = [None, None]
        gathers[0] = pltpu.async_copy(tok_hbm.at[idx_v.at[0]], t0, g0)
        pos_cp.wait()
        for b in range(B):
            cur = b % 2
            nxt = (b + 1) % 2
            if b + 1 < B:
                if stores[nxt] is not None:
                    stores[nxt].wait()
                gathers[nxt] = pltpu.async_copy(
                    tok_hbm.at[idx_v.at[b + 1]], tbufs[nxt], gsems[nxt])
            gathers[cur].wait()
            buf = tbufs[cur]

            def row(r, _, buf=buf):
                for k in range(nvec):
                    sl = pl.ds(k * _LANES, _LANES)
                    buf[r, sl] = buf[r, sl] * scale + pos_v[r, sl]
                return 0

            lax.fori_loop(0, s_per_w, row, 0)
            stores[cur] = pltpu.async_copy(
                buf, out_hbm.at[pl.ds(b * S + s0, s_per_w)], osems[cur])
        stores[0].wait()
        stores[1].wait()

    out = body(input_ids.reshape(N), token_table, pos_table)
    return out.reshape(B, S, D)
